# baseline (device time: 778578 ns/iter reference)
import jax
import jax.numpy as jnp
from jax import lax
from jax.experimental import pallas as pl
from jax.experimental.pallas import tpu as pltpu

N_DEV = 16
M = 4096
N = 2048
CHUNK = M // N_DEV


def kernel(x, w_mat, scale_x, scale_w):
    m, k_per = x.shape
    _, n = w_mat.shape

    def body(x_ref, w_ref, sx_ref, sw_ref, out_ref,
             comm_ref, send_sems, recv_sems, credit_sem):
        my = lax.axis_index("i")
        left = lax.rem(my + N_DEV - 1, N_DEV)
        right = lax.rem(my + 1, N_DEV)

        for c in range(N_DEV):
            acc = jnp.dot(
                x_ref[pl.ds(c * CHUNK, CHUNK), :],
                w_ref[:, :],
                preferred_element_type=jnp.int32,
            )
            out_ref[pl.ds(c * CHUNK, CHUNK), :] = acc.astype(jnp.float32)

        barrier_sem = pltpu.get_barrier_semaphore()
        pl.semaphore_signal(barrier_sem, inc=1, device_id=(left,),
                            device_id_type=pl.DeviceIdType.MESH)
        pl.semaphore_signal(barrier_sem, inc=1, device_id=(right,),
                            device_id_type=pl.DeviceIdType.MESH)
        pl.semaphore_wait(barrier_sem, 2)

        for s in range(N_DEV - 1):
            slot = s % 2
            if s >= 2:
                pl.semaphore_wait(credit_sem, 1)
            c_send = lax.rem(my - s + N_DEV, N_DEV)
            c_recv = lax.rem(my - s - 1 + 2 * N_DEV, N_DEV)
            rdma = pltpu.make_async_remote_copy(
                src_ref=out_ref.at[pl.ds(c_send * CHUNK, CHUNK), :],
                dst_ref=comm_ref.at[slot],
                send_sem=send_sems.at[slot],
                recv_sem=recv_sems.at[slot],
                device_id=(right,),
                device_id_type=pl.DeviceIdType.MESH,
            )
            rdma.start()
            rdma.wait()
            out_ref[pl.ds(c_recv * CHUNK, CHUNK), :] = (
                out_ref[pl.ds(c_recv * CHUNK, CHUNK), :] + comm_ref[slot]
            )
            pl.semaphore_signal(credit_sem, inc=1, device_id=(left,),
                                device_id_type=pl.DeviceIdType.MESH)
        pl.semaphore_wait(credit_sem, 2)

        sc = sx_ref[0] * sw_ref[0]
        owned = lax.rem(my + 1, N_DEV)
        off = owned * CHUNK
        y = out_ref[pl.ds(off, CHUNK), :] * sc
        out_ref[pl.ds(off, CHUNK), :] = y * (1.0 / (1.0 + jnp.exp(-y)))

        for s in range(N_DEV - 1):
            slot = s % 2
            if s >= 2:
                pl.semaphore_wait(credit_sem, 1)
            c_send = lax.rem(my + 1 - s + N_DEV, N_DEV)
            c_recv = lax.rem(my - s + 2 * N_DEV, N_DEV)
            rdma = pltpu.make_async_remote_copy(
                src_ref=out_ref.at[pl.ds(c_send * CHUNK, CHUNK), :],
                dst_ref=out_ref.at[pl.ds(c_send * CHUNK, CHUNK), :],
                send_sem=send_sems.at[slot],
                recv_sem=recv_sems.at[slot],
                device_id=(right,),
                device_id_type=pl.DeviceIdType.MESH,
            )
            rdma.start()
            rdma.wait()
            pl.semaphore_signal(credit_sem, inc=1, device_id=(left,),
                                device_id_type=pl.DeviceIdType.MESH)
        pl.semaphore_wait(credit_sem, 2)

    return pl.pallas_call(
        body,
        out_shape=jax.ShapeDtypeStruct((m, n), jnp.float32),
        in_specs=[
            pl.BlockSpec(memory_space=pltpu.VMEM),
            pl.BlockSpec(memory_space=pltpu.VMEM),
            pl.BlockSpec(memory_space=pltpu.SMEM),
            pl.BlockSpec(memory_space=pltpu.SMEM),
        ],
        out_specs=pl.BlockSpec(memory_space=pltpu.VMEM),
        scratch_shapes=[
            pltpu.VMEM((2, CHUNK, N), jnp.float32),
            pltpu.SemaphoreType.DMA((2,)),
            pltpu.SemaphoreType.DMA((2,)),
            pltpu.SemaphoreType.REGULAR,
        ],
        compiler_params=pltpu.CompilerParams(
            collective_id=0,
            vmem_limit_bytes=60 * 1024 * 1024,
        ),
    )(x, w_mat, scale_x, scale_w)


# device time: 483399 ns/iter; 1.6106x vs baseline; 1.6106x over previous
import jax
import jax.numpy as jnp
from jax import lax
from jax.experimental import pallas as pl
from jax.experimental.pallas import tpu as pltpu

N_DEV = 16
M = 4096
N = 2048
CHUNK = M // N_DEV
HALF = CHUNK // 2


def kernel(x, w_mat, scale_x, scale_w):
    m, k_per = x.shape
    _, n = w_mat.shape

    def body(x_ref, w_ref, sx_ref, sw_ref, out_ref,
             comm_p, comm_m, sems_p, sems_m, credit_p, credit_m):
        my = lax.axis_index("i")
        left = lax.rem(my + N_DEV - 1, N_DEV)
        right = lax.rem(my + 1, N_DEV)

        for c in range(N_DEV):
            acc = jnp.dot(
                x_ref[pl.ds(c * CHUNK, CHUNK), :],
                w_ref[:, :],
                preferred_element_type=jnp.int32,
            )
            out_ref[pl.ds(c * CHUNK, CHUNK), :] = acc.astype(jnp.float32)

        barrier_sem = pltpu.get_barrier_semaphore()
        pl.semaphore_signal(barrier_sem, inc=1, device_id=(left,),
                            device_id_type=pl.DeviceIdType.MESH)
        pl.semaphore_signal(barrier_sem, inc=1, device_id=(right,),
                            device_id_type=pl.DeviceIdType.MESH)
        pl.semaphore_wait(barrier_sem, 2)

        def ring_rdma(c_send, half, comm, sems, slot, target):
            return pltpu.make_async_remote_copy(
                src_ref=out_ref.at[pl.ds(c_send * CHUNK + half * HALF, HALF), :],
                dst_ref=comm.at[slot],
                send_sem=sems.at[0, slot],
                recv_sem=sems.at[1, slot],
                device_id=(target,),
                device_id_type=pl.DeviceIdType.MESH,
            )

        for s in range(N_DEV - 1):
            slot = s % 2
            if s >= 2:
                pl.semaphore_wait(credit_p, 1)
                pl.semaphore_wait(credit_m, 1)
            cs_p = lax.rem(my - s + N_DEV, N_DEV)
            cr_p = lax.rem(my - s - 1 + 2 * N_DEV, N_DEV)
            cs_m = lax.rem(my + s, N_DEV)
            cr_m = lax.rem(my + s + 1, N_DEV)
            rd_p = ring_rdma(cs_p, 0, comm_p, sems_p, slot, right)
            rd_m = ring_rdma(cs_m, 1, comm_m, sems_m, slot, left)
            rd_p.start()
            rd_m.start()
            rd_p.wait()
            rd_m.wait()
            out_ref[pl.ds(cr_p * CHUNK, HALF), :] = (
                out_ref[pl.ds(cr_p * CHUNK, HALF), :] + comm_p[slot]
            )
            out_ref[pl.ds(cr_m * CHUNK + HALF, HALF), :] = (
                out_ref[pl.ds(cr_m * CHUNK + HALF, HALF), :] + comm_m[slot]
            )
            pl.semaphore_signal(credit_p, inc=1, device_id=(left,),
                                device_id_type=pl.DeviceIdType.MESH)
            pl.semaphore_signal(credit_m, inc=1, device_id=(right,),
                                device_id_type=pl.DeviceIdType.MESH)
        pl.semaphore_wait(credit_p, 2)
        pl.semaphore_wait(credit_m, 2)

        sc = sx_ref[0] * sw_ref[0]

        def silu_inplace(off):
            y = out_ref[pl.ds(off, HALF), :] * sc
            out_ref[pl.ds(off, HALF), :] = y * (1.0 / (1.0 + jnp.exp(-y)))

        own_p = lax.rem(my + 1, N_DEV)
        own_m = lax.rem(my + N_DEV - 1, N_DEV)
        silu_inplace(own_p * CHUNK)
        silu_inplace(own_m * CHUNK + HALF)

        for s in range(N_DEV - 1):
            slot = s % 2
            if s >= 2:
                pl.semaphore_wait(credit_p, 1)
                pl.semaphore_wait(credit_m, 1)
            cs_p = lax.rem(my + 1 - s + N_DEV, N_DEV)
            cs_m = lax.rem(my - 1 + s + N_DEV, N_DEV)
            rd_p = pltpu.make_async_remote_copy(
                src_ref=out_ref.at[pl.ds(cs_p * CHUNK, HALF), :],
                dst_ref=out_ref.at[pl.ds(cs_p * CHUNK, HALF), :],
                send_sem=sems_p.at[0, slot],
                recv_sem=sems_p.at[1, slot],
                device_id=(right,),
                device_id_type=pl.DeviceIdType.MESH,
            )
            rd_m = pltpu.make_async_remote_copy(
                src_ref=out_ref.at[pl.ds(cs_m * CHUNK + HALF, HALF), :],
                dst_ref=out_ref.at[pl.ds(cs_m * CHUNK + HALF, HALF), :],
                send_sem=sems_m.at[0, slot],
                recv_sem=sems_m.at[1, slot],
                device_id=(left,),
                device_id_type=pl.DeviceIdType.MESH,
            )
            rd_p.start()
            rd_m.start()
            rd_p.wait()
            rd_m.wait()
            pl.semaphore_signal(credit_p, inc=1, device_id=(left,),
                                device_id_type=pl.DeviceIdType.MESH)
            pl.semaphore_signal(credit_m, inc=1, device_id=(right,),
                                device_id_type=pl.DeviceIdType.MESH)
        pl.semaphore_wait(credit_p, 2)
        pl.semaphore_wait(credit_m, 2)

    return pl.pallas_call(
        body,
        out_shape=jax.ShapeDtypeStruct((m, n), jnp.float32),
        in_specs=[
            pl.BlockSpec(memory_space=pltpu.VMEM),
            pl.BlockSpec(memory_space=pltpu.VMEM),
            pl.BlockSpec(memory_space=pltpu.SMEM),
            pl.BlockSpec(memory_space=pltpu.SMEM),
        ],
        out_specs=pl.BlockSpec(memory_space=pltpu.VMEM),
        scratch_shapes=[
            pltpu.VMEM((2, HALF, N), jnp.float32),
            pltpu.VMEM((2, HALF, N), jnp.float32),
            pltpu.SemaphoreType.DMA((2, 2)),
            pltpu.SemaphoreType.DMA((2, 2)),
            pltpu.SemaphoreType.REGULAR,
            pltpu.SemaphoreType.REGULAR,
        ],
        compiler_params=pltpu.CompilerParams(
            collective_id=0,
            vmem_limit_bytes=60 * 1024 * 1024,
        ),
    )(x, w_mat, scale_x, scale_w)


# device time: 428056 ns/iter; 1.8189x vs baseline; 1.1293x over previous
import jax
import jax.numpy as jnp
from jax import lax
from jax.experimental import pallas as pl
from jax.experimental.pallas import tpu as pltpu

N_DEV = 16
M = 4096
N = 2048
CHUNK = M // N_DEV
HALF = CHUNK // 2


def kernel(x, w_mat, scale_x, scale_w):
    m, k_per = x.shape
    _, n = w_mat.shape

    def body(x_ref, w_ref, sx_ref, sw_ref, out_ref,
             comm_p, comm_m, ag_comm_p, ag_comm_m, ag_own_p, ag_own_m,
             sems_p, sems_m, credit_p, credit_m):
        my = lax.axis_index("i")
        left = lax.rem(my + N_DEV - 1, N_DEV)
        right = lax.rem(my + 1, N_DEV)

        for c in range(N_DEV):
            acc = jnp.dot(
                x_ref[pl.ds(c * CHUNK, CHUNK), :],
                w_ref[:, :],
                preferred_element_type=jnp.int32,
            )
            out_ref[pl.ds(c * CHUNK, CHUNK), :] = acc.astype(jnp.float32)

        barrier_sem = pltpu.get_barrier_semaphore()
        pl.semaphore_signal(barrier_sem, inc=1, device_id=(left,),
                            device_id_type=pl.DeviceIdType.MESH)
        pl.semaphore_signal(barrier_sem, inc=1, device_id=(right,),
                            device_id_type=pl.DeviceIdType.MESH)
        pl.semaphore_wait(barrier_sem, 2)

        def ring_rdma(c_send, half, comm, sems, slot, target):
            return pltpu.make_async_remote_copy(
                src_ref=out_ref.at[pl.ds(c_send * CHUNK + half * HALF, HALF), :],
                dst_ref=comm.at[slot],
                send_sem=sems.at[0, slot],
                recv_sem=sems.at[1, slot],
                device_id=(target,),
                device_id_type=pl.DeviceIdType.MESH,
            )

        hist = {}
        for s in range(N_DEV - 1):
            slot = s % 2
            if s >= 2:
                pl.semaphore_wait(credit_p, 1)
                pl.semaphore_wait(credit_m, 1)
                hist[s - 2][0].wait_send()
                hist[s - 2][1].wait_send()
            cs_p = lax.rem(my - s + N_DEV, N_DEV)
            cr_p = lax.rem(my - s - 1 + 2 * N_DEV, N_DEV)
            cs_m = lax.rem(my + s, N_DEV)
            cr_m = lax.rem(my + s + 1, N_DEV)
            rd_p = ring_rdma(cs_p, 0, comm_p, sems_p, slot, right)
            rd_m = ring_rdma(cs_m, 1, comm_m, sems_m, slot, left)
            rd_p.start()
            rd_m.start()
            hist[s] = (rd_p, rd_m)
            rd_p.wait_recv()
            rd_m.wait_recv()
            out_ref[pl.ds(cr_p * CHUNK, HALF), :] = (
                out_ref[pl.ds(cr_p * CHUNK, HALF), :] + comm_p[slot]
            )
            out_ref[pl.ds(cr_m * CHUNK + HALF, HALF), :] = (
                out_ref[pl.ds(cr_m * CHUNK + HALF, HALF), :] + comm_m[slot]
            )
            pl.semaphore_signal(credit_p, inc=1, device_id=(left,),
                                device_id_type=pl.DeviceIdType.MESH)
            pl.semaphore_signal(credit_m, inc=1, device_id=(right,),
                                device_id_type=pl.DeviceIdType.MESH)
        hist[N_DEV - 3][0].wait_send()
        hist[N_DEV - 3][1].wait_send()
        hist[N_DEV - 2][0].wait_send()
        hist[N_DEV - 2][1].wait_send()
        pl.semaphore_wait(credit_p, 2)
        pl.semaphore_wait(credit_m, 2)

        sc = sx_ref[0] * sw_ref[0]

        def silu_stage(off, ag_own):
            y = out_ref[pl.ds(off, HALF), :] * sc
            v = y * (1.0 / (1.0 + jnp.exp(-y)))
            out_ref[pl.ds(off, HALF), :] = v
            ag_own[:, :] = v.astype(jnp.bfloat16)

        own_p = lax.rem(my + 1, N_DEV)
        own_m = lax.rem(my + N_DEV - 1, N_DEV)
        silu_stage(own_p * CHUNK, ag_own_p)
        silu_stage(own_m * CHUNK + HALF, ag_own_m)

        for s in range(N_DEV - 1):
            slot = s % 2
            if s >= 2:
                pl.semaphore_wait(credit_p, 1)
                pl.semaphore_wait(credit_m, 1)
            src_p = ag_own_p if s == 0 else ag_comm_p.at[(s - 1) % 2]
            src_m = ag_own_m if s == 0 else ag_comm_m.at[(s - 1) % 2]
            rd_p = pltpu.make_async_remote_copy(
                src_ref=src_p,
                dst_ref=ag_comm_p.at[slot],
                send_sem=sems_p.at[0, slot],
                recv_sem=sems_p.at[1, slot],
                device_id=(right,),
                device_id_type=pl.DeviceIdType.MESH,
            )
            rd_m = pltpu.make_async_remote_copy(
                src_ref=src_m,
                dst_ref=ag_comm_m.at[slot],
                send_sem=sems_m.at[0, slot],
                recv_sem=sems_m.at[1, slot],
                device_id=(left,),
                device_id_type=pl.DeviceIdType.MESH,
            )
            rd_p.start()
            rd_m.start()
            rd_p.wait_recv()
            rd_m.wait_recv()
            cr_p = lax.rem(my - s + N_DEV, N_DEV)
            cr_m = lax.rem(my + s, N_DEV)
            out_ref[pl.ds(cr_p * CHUNK, HALF), :] = (
                ag_comm_p[slot].astype(jnp.float32)
            )
            out_ref[pl.ds(cr_m * CHUNK + HALF, HALF), :] = (
                ag_comm_m[slot].astype(jnp.float32)
            )
            rd_p.wait_send()
            rd_m.wait_send()
            if s >= 1:
                pl.semaphore_signal(credit_p, inc=1, device_id=(left,),
                                    device_id_type=pl.DeviceIdType.MESH)
                pl.semaphore_signal(credit_m, inc=1, device_id=(right,),
                                    device_id_type=pl.DeviceIdType.MESH)
        pl.semaphore_wait(credit_p, 1)
        pl.semaphore_wait(credit_m, 1)

    return pl.pallas_call(
        body,
        out_shape=jax.ShapeDtypeStruct((m, n), jnp.float32),
        in_specs=[
            pl.BlockSpec(memory_space=pltpu.VMEM),
            pl.BlockSpec(memory_space=pltpu.VMEM),
            pl.BlockSpec(memory_space=pltpu.SMEM),
            pl.BlockSpec(memory_space=pltpu.SMEM),
        ],
        out_specs=pl.BlockSpec(memory_space=pltpu.VMEM),
        scratch_shapes=[
            pltpu.VMEM((2, HALF, N), jnp.float32),
            pltpu.VMEM((2, HALF, N), jnp.float32),
            pltpu.VMEM((2, HALF, N), jnp.bfloat16),
            pltpu.VMEM((2, HALF, N), jnp.bfloat16),
            pltpu.VMEM((HALF, N), jnp.bfloat16),
            pltpu.VMEM((HALF, N), jnp.bfloat16),
            pltpu.SemaphoreType.DMA((2, 2)),
            pltpu.SemaphoreType.DMA((2, 2)),
            pltpu.SemaphoreType.REGULAR,
            pltpu.SemaphoreType.REGULAR,
        ],
        compiler_params=pltpu.CompilerParams(
            collective_id=0,
            vmem_limit_bytes=60 * 1024 * 1024,
        ),
    )(x, w_mat, scale_x, scale_w)


# device time: 343596 ns/iter; 2.2660x vs baseline; 1.2458x over previous
import jax
import jax.numpy as jnp
from jax import lax
from jax.experimental import pallas as pl
from jax.experimental.pallas import tpu as pltpu

N_DEV = 16
M = 4096
N = 2048
CHUNK = M // N_DEV
HALF = CHUNK // 2


def kernel(x, w_mat, scale_x, scale_w):
    m, k_per = x.shape
    _, n = w_mat.shape

    def body(x_ref, w_ref, sx_ref, sw_ref, out_ref,
             comm_p, comm_m, send_p, send_m, ag_own_p, ag_own_m,
             sems_p, sems_m, credit_p, credit_m):
        my = lax.axis_index("i")
        left = lax.rem(my + N_DEV - 1, N_DEV)
        right = lax.rem(my + 1, N_DEV)

        for c in range(N_DEV):
            acc = jnp.dot(
                x_ref[pl.ds(c * CHUNK, CHUNK), :],
                w_ref[:, :],
                preferred_element_type=jnp.int32,
            )
            out_ref[pl.ds(c * CHUNK, CHUNK), :] = acc.astype(jnp.float32)

        send_p[0] = out_ref[pl.ds(my * CHUNK, HALF), :].astype(jnp.bfloat16)
        send_m[0] = (
            out_ref[pl.ds(my * CHUNK + HALF, HALF), :].astype(jnp.bfloat16)
        )

        barrier_sem = pltpu.get_barrier_semaphore()
        pl.semaphore_signal(barrier_sem, inc=1, device_id=(left,),
                            device_id_type=pl.DeviceIdType.MESH)
        pl.semaphore_signal(barrier_sem, inc=1, device_id=(right,),
                            device_id_type=pl.DeviceIdType.MESH)
        pl.semaphore_wait(barrier_sem, 2)

        hist = {}
        for s in range(N_DEV - 1):
            slot = s % 2
            nslot = (s + 1) % 2
            if s >= 2:
                pl.semaphore_wait(credit_p, 1)
                pl.semaphore_wait(credit_m, 1)
            cr_p = lax.rem(my - s - 1 + 2 * N_DEV, N_DEV)
            cr_m = lax.rem(my + s + 1, N_DEV)
            rd_p = pltpu.make_async_remote_copy(
                src_ref=send_p.at[slot],
                dst_ref=comm_p.at[slot],
                send_sem=sems_p.at[0, slot],
                recv_sem=sems_p.at[1, slot],
                device_id=(right,),
                device_id_type=pl.DeviceIdType.MESH,
            )
            rd_m = pltpu.make_async_remote_copy(
                src_ref=send_m.at[slot],
                dst_ref=comm_m.at[slot],
                send_sem=sems_m.at[0, slot],
                recv_sem=sems_m.at[1, slot],
                device_id=(left,),
                device_id_type=pl.DeviceIdType.MESH,
            )
            rd_p.start()
            rd_m.start()
            hist[s] = (rd_p, rd_m)
            rd_p.wait_recv()
            rd_m.wait_recv()
            if s >= 1:
                hist[s - 1][0].wait_send()
                hist[s - 1][1].wait_send()
            acc_p = (
                out_ref[pl.ds(cr_p * CHUNK, HALF), :]
                + comm_p[slot].astype(jnp.float32)
            )
            out_ref[pl.ds(cr_p * CHUNK, HALF), :] = acc_p
            acc_m = (
                out_ref[pl.ds(cr_m * CHUNK + HALF, HALF), :]
                + comm_m[slot].astype(jnp.float32)
            )
            out_ref[pl.ds(cr_m * CHUNK + HALF, HALF), :] = acc_m
            if s < N_DEV - 2:
                send_p[nslot] = acc_p.astype(jnp.bfloat16)
                send_m[nslot] = acc_m.astype(jnp.bfloat16)
            pl.semaphore_signal(credit_p, inc=1, device_id=(left,),
                                device_id_type=pl.DeviceIdType.MESH)
            pl.semaphore_signal(credit_m, inc=1, device_id=(right,),
                                device_id_type=pl.DeviceIdType.MESH)
        hist[N_DEV - 2][0].wait_send()
        hist[N_DEV - 2][1].wait_send()
        pl.semaphore_wait(credit_p, 2)
        pl.semaphore_wait(credit_m, 2)

        sc = sx_ref[0] * sw_ref[0]

        def silu_stage(off, ag_own):
            y = out_ref[pl.ds(off, HALF), :] * sc
            v = y * (1.0 / (1.0 + jnp.exp(-y)))
            out_ref[pl.ds(off, HALF), :] = v
            ag_own[:, :] = v.astype(jnp.bfloat16)

        own_p = lax.rem(my + 1, N_DEV)
        own_m = lax.rem(my + N_DEV - 1, N_DEV)
        silu_stage(own_p * CHUNK, ag_own_p)
        silu_stage(own_m * CHUNK + HALF, ag_own_m)

        for s in range(N_DEV - 1):
            slot = s % 2
            pslot = (s - 1) % 2
            if s >= 2:
                pl.semaphore_wait(credit_p, 1)
                pl.semaphore_wait(credit_m, 1)
            src_p = ag_own_p if s == 0 else comm_p.at[pslot]
            src_m = ag_own_m if s == 0 else comm_m.at[pslot]
            rd_p = pltpu.make_async_remote_copy(
                src_ref=src_p,
                dst_ref=comm_p.at[slot],
                send_sem=sems_p.at[0, slot],
                recv_sem=sems_p.at[1, slot],
                device_id=(right,),
                device_id_type=pl.DeviceIdType.MESH,
            )
            rd_m = pltpu.make_async_remote_copy(
                src_ref=src_m,
                dst_ref=comm_m.at[slot],
                send_sem=sems_m.at[0, slot],
                recv_sem=sems_m.at[1, slot],
                device_id=(left,),
                device_id_type=pl.DeviceIdType.MESH,
            )
            rd_p.start()
            rd_m.start()
            if s >= 1:
                cr_p = lax.rem(my - s + 1 + N_DEV, N_DEV)
                cr_m = lax.rem(my + s - 1, N_DEV)
                out_ref[pl.ds(cr_p * CHUNK, HALF), :] = (
                    comm_p[pslot].astype(jnp.float32)
                )
                out_ref[pl.ds(cr_m * CHUNK + HALF, HALF), :] = (
                    comm_m[pslot].astype(jnp.float32)
                )
            rd_p.wait_recv()
            rd_m.wait_recv()
            rd_p.wait_send()
            rd_m.wait_send()
            if s >= 1:
                pl.semaphore_signal(credit_p, inc=1, device_id=(left,),
                                    device_id_type=pl.DeviceIdType.MESH)
                pl.semaphore_signal(credit_m, inc=1, device_id=(right,),
                                    device_id_type=pl.DeviceIdType.MESH)
        cr_p = lax.rem(my - N_DEV + 2 + N_DEV, N_DEV)
        cr_m = lax.rem(my + N_DEV - 2, N_DEV)
        out_ref[pl.ds(cr_p * CHUNK, HALF), :] = (
            comm_p[(N_DEV - 2) % 2].astype(jnp.float32)
        )
        out_ref[pl.ds(cr_m * CHUNK + HALF, HALF), :] = (
            comm_m[(N_DEV - 2) % 2].astype(jnp.float32)
        )
        pl.semaphore_wait(credit_p, 1)
        pl.semaphore_wait(credit_m, 1)

    return pl.pallas_call(
        body,
        out_shape=jax.ShapeDtypeStruct((m, n), jnp.float32),
        in_specs=[
            pl.BlockSpec(memory_space=pltpu.VMEM),
            pl.BlockSpec(memory_space=pltpu.VMEM),
            pl.BlockSpec(memory_space=pltpu.SMEM),
            pl.BlockSpec(memory_space=pltpu.SMEM),
        ],
        out_specs=pl.BlockSpec(memory_space=pltpu.VMEM),
        scratch_shapes=[
            pltpu.VMEM((2, HALF, N), jnp.bfloat16),
            pltpu.VMEM((2, HALF, N), jnp.bfloat16),
            pltpu.VMEM((2, HALF, N), jnp.bfloat16),
            pltpu.VMEM((2, HALF, N), jnp.bfloat16),
            pltpu.VMEM((HALF, N), jnp.bfloat16),
            pltpu.VMEM((HALF, N), jnp.bfloat16),
            pltpu.SemaphoreType.DMA((2, 2)),
            pltpu.SemaphoreType.DMA((2, 2)),
            pltpu.SemaphoreType.REGULAR,
            pltpu.SemaphoreType.REGULAR,
        ],
        compiler_params=pltpu.CompilerParams(
            collective_id=0,
            vmem_limit_bytes=60 * 1024 * 1024,
        ),
    )(x, w_mat, scale_x, scale_w)


# device time: 339697 ns/iter; 2.2920x vs baseline; 1.0115x over previous
import jax
import jax.numpy as jnp
from jax import lax
from jax.experimental import pallas as pl
from jax.experimental.pallas import tpu as pltpu

N_DEV = 16
M = 4096
N = 2048
CHUNK = M // N_DEV
QTR = CHUNK // 4


def kernel(x, w_mat, scale_x, scale_w):
    m, k_per = x.shape
    _, n = w_mat.shape

    def body(x_ref, w_ref, sx_ref, sw_ref, out_ref, *scratch):
        comms = scratch[0:4]
        sends = scratch[4:8]
        owns = scratch[8:12]
        sems = scratch[12:16]
        credits = scratch[16:20]

        my = lax.axis_index("i")
        left = lax.rem(my + N_DEV - 1, N_DEV)
        right = lax.rem(my + 1, N_DEV)

        rings = [
            (0, 0 * QTR, right, left, +1),
            (2, 2 * QTR, left, right, -1),
            (1, 1 * QTR, right, left, +1),
            (3, 3 * QTR, left, right, -1),
        ]

        def rows(c, base):
            return pl.ds(c * CHUNK + base, QTR)

        def c_send(s, sign):
            return lax.rem(my - sign * s + 2 * N_DEV, N_DEV)

        def c_recv(s, sign):
            return lax.rem(my - sign * (s + 1) + 2 * N_DEV, N_DEV)

        for c in range(N_DEV):
            acc = jnp.dot(
                x_ref[pl.ds(c * CHUNK, CHUNK), :],
                w_ref[:, :],
                preferred_element_type=jnp.int32,
            )
            out_ref[pl.ds(c * CHUNK, CHUNK), :] = acc.astype(jnp.float32)

        for i, base, _, _, _ in rings:
            sends[i][0] = out_ref[rows(my, base), :].astype(jnp.bfloat16)

        barrier_sem = pltpu.get_barrier_semaphore()
        pl.semaphore_signal(barrier_sem, inc=1, device_id=(left,),
                            device_id_type=pl.DeviceIdType.MESH)
        pl.semaphore_signal(barrier_sem, inc=1, device_id=(right,),
                            device_id_type=pl.DeviceIdType.MESH)
        pl.semaphore_wait(barrier_sem, 2)

        hist = {}
        for s in range(N_DEV - 1):
            slot = s % 2
            nslot = (s + 1) % 2
            if s >= 2:
                for cr_sem in credits:
                    pl.semaphore_wait(cr_sem, 1)
            for i, base, to, _, sign in rings:
                rd = pltpu.make_async_remote_copy(
                    src_ref=sends[i].at[slot],
                    dst_ref=comms[i].at[slot],
                    send_sem=sems[i].at[0, slot],
                    recv_sem=sems[i].at[1, slot],
                    device_id=(to,),
                    device_id_type=pl.DeviceIdType.MESH,
                )
                rd.start()
                hist[(i, s)] = rd
            for i, base, to, cto, sign in rings:
                hist[(i, s)].wait_recv()
                if s >= 1:
                    hist[(i, s - 1)].wait_send()
                cr = c_recv(s, sign)
                acc = (
                    out_ref[rows(cr, base), :]
                    + comms[i][slot].astype(jnp.float32)
                )
                out_ref[rows(cr, base), :] = acc
                if s < N_DEV - 2:
                    sends[i][nslot] = acc.astype(jnp.bfloat16)
                pl.semaphore_signal(credits[i], inc=1, device_id=(cto,),
                                    device_id_type=pl.DeviceIdType.MESH)
        for i, *_ in rings:
            hist[(i, N_DEV - 2)].wait_send()
        for cr_sem in credits:
            pl.semaphore_wait(cr_sem, 2)

        sc = sx_ref[0] * sw_ref[0]
        for i, base, _, _, sign in rings:
            own = lax.rem(my + sign + N_DEV, N_DEV)
            y = out_ref[rows(own, base), :] * sc
            v = y * (1.0 / (1.0 + jnp.exp(-y)))
            out_ref[rows(own, base), :] = v
            owns[i][:, :] = v.astype(jnp.bfloat16)

        for s in range(N_DEV - 1):
            slot = s % 2
            pslot = (s - 1) % 2
            if s >= 2:
                for cr_sem in credits:
                    pl.semaphore_wait(cr_sem, 1)
            for i, base, to, _, sign in rings:
                rd = pltpu.make_async_remote_copy(
                    src_ref=owns[i] if s == 0 else comms[i].at[pslot],
                    dst_ref=comms[i].at[slot],
                    send_sem=sems[i].at[0, slot],
                    recv_sem=sems[i].at[1, slot],
                    device_id=(to,),
                    device_id_type=pl.DeviceIdType.MESH,
                )
                rd.start()
                hist[(i, s)] = rd
            if s >= 1:
                for i, base, _, _, sign in rings:
                    cg = c_send(s - 1, sign)
                    out_ref[rows(cg, base), :] = (
                        comms[i][pslot].astype(jnp.float32)
                    )
            for i, base, _, cto, sign in rings:
                hist[(i, s)].wait_recv()
                hist[(i, s)].wait_send()
                if s >= 1:
                    pl.semaphore_signal(credits[i], inc=1, device_id=(cto,),
                                        device_id_type=pl.DeviceIdType.MESH)
        for i, base, _, _, sign in rings:
            cg = c_send(N_DEV - 2, sign)
            out_ref[rows(cg, base), :] = (
                comms[i][(N_DEV - 2) % 2].astype(jnp.float32)
            )
        for cr_sem in credits:
            pl.semaphore_wait(cr_sem, 1)

    return pl.pallas_call(
        body,
        out_shape=jax.ShapeDtypeStruct((m, n), jnp.float32),
        in_specs=[
            pl.BlockSpec(memory_space=pltpu.VMEM),
            pl.BlockSpec(memory_space=pltpu.VMEM),
            pl.BlockSpec(memory_space=pltpu.SMEM),
            pl.BlockSpec(memory_space=pltpu.SMEM),
        ],
        out_specs=pl.BlockSpec(memory_space=pltpu.VMEM),
        scratch_shapes=(
            [pltpu.VMEM((2, QTR, N), jnp.bfloat16)] * 4
            + [pltpu.VMEM((2, QTR, N), jnp.bfloat16)] * 4
            + [pltpu.VMEM((QTR, N), jnp.bfloat16)] * 4
            + [pltpu.SemaphoreType.DMA((2, 2))] * 4
            + [pltpu.SemaphoreType.REGULAR] * 4
        ),
        compiler_params=pltpu.CompilerParams(
            collective_id=0,
            vmem_limit_bytes=60 * 1024 * 1024,
        ),
    )(x, w_mat, scale_x, scale_w)


# device time: 292228 ns/iter; 2.6643x vs baseline; 1.1624x over previous
import jax
import jax.numpy as jnp
from jax import lax
from jax.experimental import pallas as pl
from jax.experimental.pallas import tpu as pltpu

N_DEV = 16
M = 4096
N = 2048
CHUNK = M // N_DEV
QTR = CHUNK // 4


def kernel(x, w_mat, scale_x, scale_w):
    m, k_per = x.shape
    _, n = w_mat.shape

    def body(x_ref, w_ref, sx_ref, sw_ref, out_ref, *scratch):
        comms = scratch[0:4]
        sends = scratch[4:8]
        owns = scratch[8:12]
        sems = scratch[12:16]
        credits = scratch[16:20]

        my = lax.axis_index("i")
        left = lax.rem(my + N_DEV - 1, N_DEV)
        right = lax.rem(my + 1, N_DEV)

        rings = [
            (0, 0 * QTR, right, left, +1),
            (2, 2 * QTR, left, right, -1),
            (1, 1 * QTR, right, left, +1),
            (3, 3 * QTR, left, right, -1),
        ]

        def rows(c, base):
            return pl.ds(c * CHUNK + base, QTR)

        def c_send(s, sign):
            return lax.rem(my - sign * s + 2 * N_DEV, N_DEV)

        def c_recv(s, sign):
            return lax.rem(my - sign * (s + 1) + 2 * N_DEV, N_DEV)

        def gemm_chunk(delta):
            c = lax.rem(my + delta + N_DEV, N_DEV)
            acc = jnp.dot(
                x_ref[pl.ds(c * CHUNK, CHUNK), :],
                w_ref[:, :],
                preferred_element_type=jnp.int32,
            )
            out_ref[pl.ds(c * CHUNK, CHUNK), :] = acc.astype(jnp.float32)

        def rs_rdma(i, slot, to):
            return pltpu.make_async_remote_copy(
                src_ref=sends[i].at[slot],
                dst_ref=comms[i].at[slot],
                send_sem=sems[i].at[0, slot],
                recv_sem=sems[i].at[1, slot],
                device_id=(to,),
                device_id_type=pl.DeviceIdType.MESH,
            )

        for d in (0, 1, -1):
            gemm_chunk(d)

        for i, base, _, _, _ in rings:
            sends[i][0] = out_ref[rows(my, base), :].astype(jnp.bfloat16)

        barrier_sem = pltpu.get_barrier_semaphore()
        pl.semaphore_signal(barrier_sem, inc=1, device_id=(left,),
                            device_id_type=pl.DeviceIdType.MESH)
        pl.semaphore_signal(barrier_sem, inc=1, device_id=(right,),
                            device_id_type=pl.DeviceIdType.MESH)
        pl.semaphore_wait(barrier_sem, 2)

        hist = {}
        for i, base, to, _, sign in rings:
            rd = rs_rdma(i, 0, to)
            rd.start()
            hist[(i, 0)] = rd
        for d in (2, -2, 3, -3, 4, -4, 5, -5, 6, -6, 7, -7, 8):
            gemm_chunk(d)

        for s in range(N_DEV - 1):
            slot = s % 2
            nslot = (s + 1) % 2
            for i, base, to, cto, sign in rings:
                hist[(i, s)].wait_recv()
                if s >= 1:
                    hist[(i, s - 1)].wait_send()
                cr = c_recv(s, sign)
                acc = (
                    out_ref[rows(cr, base), :]
                    + comms[i][slot].astype(jnp.float32)
                )
                out_ref[rows(cr, base), :] = acc
                if s < N_DEV - 2:
                    sends[i][nslot] = acc.astype(jnp.bfloat16)
                    if s >= 1:
                        pl.semaphore_wait(credits[i], 1)
                    rd = rs_rdma(i, nslot, to)
                    rd.start()
                    hist[(i, s + 1)] = rd
                pl.semaphore_signal(credits[i], inc=1, device_id=(cto,),
                                    device_id_type=pl.DeviceIdType.MESH)
        for i, *_ in rings:
            hist[(i, N_DEV - 2)].wait_send()
        for cr_sem in credits:
            pl.semaphore_wait(cr_sem, 2)

        sc = sx_ref[0] * sw_ref[0]
        for i, base, _, _, sign in rings:
            own = lax.rem(my + sign + N_DEV, N_DEV)
            y = out_ref[rows(own, base), :] * sc
            v = y * (1.0 / (1.0 + jnp.exp(-y)))
            out_ref[rows(own, base), :] = v
            owns[i][:, :] = v.astype(jnp.bfloat16)

        for s in range(N_DEV - 1):
            slot = s % 2
            pslot = (s - 1) % 2
            if s >= 2:
                for cr_sem in credits:
                    pl.semaphore_wait(cr_sem, 1)
            for i, base, to, _, sign in rings:
                rd = pltpu.make_async_remote_copy(
                    src_ref=owns[i] if s == 0 else comms[i].at[pslot],
                    dst_ref=comms[i].at[slot],
                    send_sem=sems[i].at[0, slot],
                    recv_sem=sems[i].at[1, slot],
                    device_id=(to,),
                    device_id_type=pl.DeviceIdType.MESH,
                )
                rd.start()
                hist[(i, s)] = rd
            if s >= 1:
                for i, base, _, _, sign in rings:
                    cg = c_send(s - 1, sign)
                    out_ref[rows(cg, base), :] = (
                        comms[i][pslot].astype(jnp.float32)
                    )
            for i, base, _, cto, sign in rings:
                hist[(i, s)].wait_recv()
                hist[(i, s)].wait_send()
                if s >= 1:
                    pl.semaphore_signal(credits[i], inc=1, device_id=(cto,),
                                        device_id_type=pl.DeviceIdType.MESH)
        for i, base, _, _, sign in rings:
            cg = c_send(N_DEV - 2, sign)
            out_ref[rows(cg, base), :] = (
                comms[i][(N_DEV - 2) % 2].astype(jnp.float32)
            )
        for cr_sem in credits:
            pl.semaphore_wait(cr_sem, 1)

    return pl.pallas_call(
        body,
        out_shape=jax.ShapeDtypeStruct((m, n), jnp.float32),
        in_specs=[
            pl.BlockSpec(memory_space=pltpu.VMEM),
            pl.BlockSpec(memory_space=pltpu.VMEM),
            pl.BlockSpec(memory_space=pltpu.SMEM),
            pl.BlockSpec(memory_space=pltpu.SMEM),
        ],
        out_specs=pl.BlockSpec(memory_space=pltpu.VMEM),
        scratch_shapes=(
            [pltpu.VMEM((2, QTR, N), jnp.bfloat16)] * 4
            + [pltpu.VMEM((2, QTR, N), jnp.bfloat16)] * 4
            + [pltpu.VMEM((QTR, N), jnp.bfloat16)] * 4
            + [pltpu.SemaphoreType.DMA((2, 2))] * 4
            + [pltpu.SemaphoreType.REGULAR] * 4
        ),
        compiler_params=pltpu.CompilerParams(
            collective_id=0,
            vmem_limit_bytes=60 * 1024 * 1024,
        ),
    )(x, w_mat, scale_x, scale_w)


# device time: 222914 ns/iter; 3.4927x vs baseline; 1.3109x over previous
import jax
import jax.numpy as jnp
from jax import lax
from jax.experimental import pallas as pl
from jax.experimental.pallas import tpu as pltpu

N_DEV = 16
M = 4096
N = 2048
CHUNK = M // N_DEV
QTR = CHUNK // 4


def kernel(x, w_mat, scale_x, scale_w):
    m, k_per = x.shape
    _, n = w_mat.shape

    def body(x_ref, w_ref, sx_ref, sw_ref, out_ref, *scratch):
        comms = scratch[0:4]
        sends = scratch[4:8]
        owns = scratch[8:12]
        sems = scratch[12:16]
        credits = scratch[16:20]

        my = lax.axis_index("i")
        left = lax.rem(my + N_DEV - 1, N_DEV)
        right = lax.rem(my + 1, N_DEV)

        rings = [
            (0, 0 * QTR, right, left, +1),
            (2, 2 * QTR, left, right, -1),
            (1, 1 * QTR, right, left, +1),
            (3, 3 * QTR, left, right, -1),
        ]

        def rows(c, base):
            return pl.ds(c * CHUNK + base, QTR)

        def c_send(s, sign):
            return lax.rem(my - sign * s + 2 * N_DEV, N_DEV)

        def c_recv(s, sign):
            return lax.rem(my - sign * (s + 1) + 2 * N_DEV, N_DEV)

        def gemm_chunk(delta):
            c = lax.rem(my + delta + N_DEV, N_DEV)
            acc = jnp.dot(
                x_ref[pl.ds(c * CHUNK, CHUNK), :],
                w_ref[:, :],
                preferred_element_type=jnp.int32,
            )
            out_ref[pl.ds(c * CHUNK, CHUNK), :] = acc.astype(jnp.float32)

        def rs_rdma(i, slot, to):
            return pltpu.make_async_remote_copy(
                src_ref=sends[i].at[slot],
                dst_ref=comms[i].at[slot],
                send_sem=sems[i].at[0, slot],
                recv_sem=sems[i].at[1, slot],
                device_id=(to,),
                device_id_type=pl.DeviceIdType.MESH,
            )

        for d in (0, 1, -1):
            gemm_chunk(d)

        for i, base, _, _, _ in rings:
            sends[i][0] = out_ref[rows(my, base), :].astype(jnp.bfloat16)

        barrier_sem = pltpu.get_barrier_semaphore()
        pl.semaphore_signal(barrier_sem, inc=1, device_id=(left,),
                            device_id_type=pl.DeviceIdType.MESH)
        pl.semaphore_signal(barrier_sem, inc=1, device_id=(right,),
                            device_id_type=pl.DeviceIdType.MESH)
        pl.semaphore_wait(barrier_sem, 2)

        hist = {}
        for i, base, to, _, sign in rings:
            rd = rs_rdma(i, 0, to)
            rd.start()
            hist[(i, 0)] = rd
        for d in (2, -2, 3, -3, 4, -4, 5, -5, 6, -6, 7, -7, 8):
            gemm_chunk(d)

        for s in range(N_DEV - 1):
            slot = s % 2
            nslot = (s + 1) % 2
            for i, base, to, cto, sign in rings:
                hist[(i, s)].wait_recv()
                if s >= 1:
                    hist[(i, s - 1)].wait_send()
                cr = c_recv(s, sign)
                acc = (
                    out_ref[rows(cr, base), :]
                    + comms[i][slot].astype(jnp.float32)
                )
                out_ref[rows(cr, base), :] = acc
                if s < N_DEV - 2:
                    sends[i][nslot] = acc.astype(jnp.bfloat16)
                    if s >= 1:
                        pl.semaphore_wait(credits[i], 1)
                    rd = rs_rdma(i, nslot, to)
                    rd.start()
                    hist[(i, s + 1)] = rd
                pl.semaphore_signal(credits[i], inc=1, device_id=(cto,),
                                    device_id_type=pl.DeviceIdType.MESH)
        for i, *_ in rings:
            hist[(i, N_DEV - 2)].wait_send()
        for cr_sem in credits:
            pl.semaphore_wait(cr_sem, 2)

        sc = sx_ref[0] * sw_ref[0]
        for i, base, _, _, sign in rings:
            own = lax.rem(my + sign + N_DEV, N_DEV)
            y = out_ref[rows(own, base), :] * sc
            v = y * (1.0 / (1.0 + jnp.exp(-y)))
            out_ref[rows(own, base), :] = v
            owns[i][:, :] = v.astype(jnp.bfloat16)

        def ag_rdma(i, src, slot, to):
            return pltpu.make_async_remote_copy(
                src_ref=src,
                dst_ref=comms[i].at[slot],
                send_sem=sems[i].at[0, slot],
                recv_sem=sems[i].at[1, slot],
                device_id=(to,),
                device_id_type=pl.DeviceIdType.MESH,
            )

        for i, base, to, _, sign in rings:
            rd = ag_rdma(i, owns[i], 0, to)
            rd.start()
            hist[(i, 0)] = rd
        for s in range(N_DEV - 1):
            slot = s % 4
            nslot = (s + 1) % 4
            for i, base, to, cto, sign in rings:
                hist[(i, s)].wait_recv()
                if s < N_DEV - 2:
                    if s >= 2:
                        pl.semaphore_wait(credits[i], 1)
                    rd = ag_rdma(i, comms[i].at[slot], nslot, to)
                    rd.start()
                    hist[(i, s + 1)] = rd
                cg = c_send(s, sign)
                out_ref[rows(cg, base), :] = (
                    comms[i][slot].astype(jnp.float32)
                )
                hist[(i, s)].wait_send()
                pl.semaphore_signal(credits[i], inc=1, device_id=(cto,),
                                    device_id_type=pl.DeviceIdType.MESH)
        for cr_sem in credits:
            pl.semaphore_wait(cr_sem, 3)

    return pl.pallas_call(
        body,
        out_shape=jax.ShapeDtypeStruct((m, n), jnp.float32),
        in_specs=[
            pl.BlockSpec(memory_space=pltpu.VMEM),
            pl.BlockSpec(memory_space=pltpu.VMEM),
            pl.BlockSpec(memory_space=pltpu.SMEM),
            pl.BlockSpec(memory_space=pltpu.SMEM),
        ],
        out_specs=pl.BlockSpec(memory_space=pltpu.VMEM),
        scratch_shapes=(
            [pltpu.VMEM((4, QTR, N), jnp.bfloat16)] * 4
            + [pltpu.VMEM((2, QTR, N), jnp.bfloat16)] * 4
            + [pltpu.VMEM((QTR, N), jnp.bfloat16)] * 4
            + [pltpu.SemaphoreType.DMA((2, 4))] * 4
            + [pltpu.SemaphoreType.REGULAR] * 4
        ),
        compiler_params=pltpu.CompilerParams(
            collective_id=0,
            vmem_limit_bytes=60 * 1024 * 1024,
        ),
    )(x, w_mat, scale_x, scale_w)


# device time: 222444 ns/iter; 3.5001x vs baseline; 1.0021x over previous
import jax
import jax.numpy as jnp
from jax import lax
from jax.experimental import pallas as pl
from jax.experimental.pallas import tpu as pltpu

N_DEV = 16
M = 4096
N = 2048
CHUNK = M // N_DEV
QTR = CHUNK // 4


def kernel(x, w_mat, scale_x, scale_w):
    m, k_per = x.shape
    _, n = w_mat.shape

    def body(x_ref, w_ref, sx_ref, sw_ref, out_ref, *scratch):
        comms = scratch[0:4]
        sends = scratch[4:8]
        owns = scratch[8:12]
        sems = scratch[12:16]
        credits = scratch[16:20]

        my = lax.axis_index("i")
        left = lax.rem(my + N_DEV - 1, N_DEV)
        right = lax.rem(my + 1, N_DEV)

        rings = [
            (0, 0 * QTR, right, left, +1),
            (2, 2 * QTR, left, right, -1),
            (1, 1 * QTR, right, left, +1),
            (3, 3 * QTR, left, right, -1),
        ]

        def rows(c, base):
            return pl.ds(c * CHUNK + base, QTR)

        def c_send(s, sign):
            return lax.rem(my - sign * s + 2 * N_DEV, N_DEV)

        def c_recv(s, sign):
            return lax.rem(my - sign * (s + 1) + 2 * N_DEV, N_DEV)

        def gemm_chunk(delta):
            c = lax.rem(my + delta + N_DEV, N_DEV)
            acc = jnp.dot(
                x_ref[pl.ds(c * CHUNK, CHUNK), :],
                w_ref[:, :],
                preferred_element_type=jnp.int32,
            )
            out_ref[pl.ds(c * CHUNK, CHUNK), :] = acc.astype(jnp.float32)

        def rs_rdma(i, slot, to):
            return pltpu.make_async_remote_copy(
                src_ref=sends[i].at[slot],
                dst_ref=comms[i].at[slot],
                send_sem=sems[i].at[0, slot],
                recv_sem=sems[i].at[1, slot],
                device_id=(to,),
                device_id_type=pl.DeviceIdType.MESH,
            )

        for d in (0, 1, -1):
            gemm_chunk(d)

        for i, base, _, _, _ in rings:
            sends[i][0] = out_ref[rows(my, base), :].astype(jnp.bfloat16)

        barrier_sem = pltpu.get_barrier_semaphore()
        pl.semaphore_signal(barrier_sem, inc=1, device_id=(left,),
                            device_id_type=pl.DeviceIdType.MESH)
        pl.semaphore_signal(barrier_sem, inc=1, device_id=(right,),
                            device_id_type=pl.DeviceIdType.MESH)
        pl.semaphore_wait(barrier_sem, 2)

        hist = {}
        for i, base, to, _, sign in rings:
            rd = rs_rdma(i, 0, to)
            rd.start()
            hist[(i, 0)] = rd
        for d in (2, -2, 3, -3, 4, -4, 5, -5, 6, -6, 7, -7, 8):
            gemm_chunk(d)

        for s in range(N_DEV - 1):
            slot = s % 4
            nslot = (s + 1) % 4
            for i, base, to, cto, sign in rings:
                hist[(i, s)].wait_recv()
                if s >= 3:
                    hist[(i, s - 3)].wait_send()
                cr = c_recv(s, sign)
                acc = (
                    out_ref[rows(cr, base), :]
                    + comms[i][slot].astype(jnp.float32)
                )
                out_ref[rows(cr, base), :] = acc
                if s < N_DEV - 2:
                    sends[i][nslot] = acc.astype(jnp.bfloat16)
                    if s >= 3:
                        pl.semaphore_wait(credits[i], 1)
                    rd = rs_rdma(i, nslot, to)
                    rd.start()
                    hist[(i, s + 1)] = rd
                pl.semaphore_signal(credits[i], inc=1, device_id=(cto,),
                                    device_id_type=pl.DeviceIdType.MESH)
        for i, *_ in rings:
            for u in (N_DEV - 4, N_DEV - 3, N_DEV - 2):
                hist[(i, u)].wait_send()
        for cr_sem in credits:
            pl.semaphore_wait(cr_sem, 4)

        sc = sx_ref[0] * sw_ref[0]
        for i, base, _, _, sign in rings:
            own = lax.rem(my + sign + N_DEV, N_DEV)
            y = out_ref[rows(own, base), :] * sc
            v = y * (1.0 / (1.0 + jnp.exp(-y)))
            out_ref[rows(own, base), :] = v
            owns[i][:, :] = v.astype(jnp.bfloat16)

        def ag_rdma(i, src, slot, to):
            return pltpu.make_async_remote_copy(
                src_ref=src,
                dst_ref=comms[i].at[slot],
                send_sem=sems[i].at[0, slot],
                recv_sem=sems[i].at[1, slot],
                device_id=(to,),
                device_id_type=pl.DeviceIdType.MESH,
            )

        for i, base, to, _, sign in rings:
            rd = ag_rdma(i, owns[i], 0, to)
            rd.start()
            hist[(i, 0)] = rd
        for s in range(N_DEV - 1):
            slot = s % 4
            nslot = (s + 1) % 4
            for i, base, to, cto, sign in rings:
                hist[(i, s)].wait_recv()
                if s < N_DEV - 2:
                    if s >= 2:
                        pl.semaphore_wait(credits[i], 1)
                    rd = ag_rdma(i, comms[i].at[slot], nslot, to)
                    rd.start()
                    hist[(i, s + 1)] = rd
                cg = c_send(s, sign)
                out_ref[rows(cg, base), :] = (
                    comms[i][slot].astype(jnp.float32)
                )
                hist[(i, s)].wait_send()
                pl.semaphore_signal(credits[i], inc=1, device_id=(cto,),
                                    device_id_type=pl.DeviceIdType.MESH)
        for cr_sem in credits:
            pl.semaphore_wait(cr_sem, 3)

    return pl.pallas_call(
        body,
        out_shape=jax.ShapeDtypeStruct((m, n), jnp.float32),
        in_specs=[
            pl.BlockSpec(memory_space=pltpu.VMEM),
            pl.BlockSpec(memory_space=pltpu.VMEM),
            pl.BlockSpec(memory_space=pltpu.SMEM),
            pl.BlockSpec(memory_space=pltpu.SMEM),
        ],
        out_specs=pl.BlockSpec(memory_space=pltpu.VMEM),
        scratch_shapes=(
            [pltpu.VMEM((4, QTR, N), jnp.bfloat16)] * 4
            + [pltpu.VMEM((4, QTR, N), jnp.bfloat16)] * 4
            + [pltpu.VMEM((QTR, N), jnp.bfloat16)] * 4
            + [pltpu.SemaphoreType.DMA((2, 4))] * 4
            + [pltpu.SemaphoreType.REGULAR] * 4
        ),
        compiler_params=pltpu.CompilerParams(
            collective_id=0,
            vmem_limit_bytes=60 * 1024 * 1024,
        ),
    )(x, w_mat, scale_x, scale_w)
